# baseline (speedup 1.0000x reference)
import functools

import jax
import jax.numpy as jnp
from jax import lax
from jax.experimental import pallas as pl
from jax.experimental.pallas import tpu as pltpu
from jax.experimental.pallas import tpu_sc as plsc

EMB_DIM = 128
BATCH = 4096
HIST = 50

NC = 2
NS = 16
NW = NC * NS

NCB = 16
NBAND = 2
BLOCK = BATCH // NCB
GLEN = 128
GPC = BLOCK // GLEN
BAND = HIST // NBAND
NBUF = 3
GAHEAD = 2


def _make_gather():
  mesh = plsc.VectorSubcoreMesh(core_axis_name="c", subcore_axis_name="s")

  @functools.partial(
      pl.kernel,
      mesh=mesh,
      out_type=jax.ShapeDtypeStruct((HIST, BATCH, EMB_DIM), jnp.float32),
      scratch_types=[
          pltpu.VMEM((HIST, BLOCK), jnp.int32),
          pltpu.VMEM((NBUF, BLOCK, EMB_DIM), jnp.float32),
          pltpu.SemaphoreType.DMA,
          pltpu.SemaphoreType.DMA,
      ],
      compiler_params=pltpu.CompilerParams(use_tc_tiling_on_sc=True),
  )
  def gather_kernel(table_hbm, idx_hbm, out_hbm, idx_v, rows_v, gsem, ssem):
    wid = lax.axis_index("s") * NC + lax.axis_index("c")
    cb = lax.rem(wid, NCB)
    band = wid // NCB
    col = cb * BLOCK
    h0 = band * BAND
    skip = h0
    pltpu.sync_copy(idx_hbm.at[:, pl.ds(col, BLOCK)], idx_v)

    def fill(j):
      b = lax.rem(j, NBUF)
      for g in range(GPC):
        pltpu.async_copy(
            table_hbm.at[idx_v.at[j + skip, pl.ds(g * GLEN, GLEN)]],
            rows_v.at[b, pl.ds(g * GLEN, GLEN)], gsem)

    def drain_gather(j):
      b = lax.rem(j, NBUF)
      for g in range(GPC):
        pltpu.make_async_copy(
            table_hbm.at[idx_v.at[j + skip, pl.ds(g * GLEN, GLEN)]],
            rows_v.at[b, pl.ds(g * GLEN, GLEN)], gsem).wait()

    def start_store(j):
      pltpu.make_async_copy(rows_v.at[lax.rem(j, NBUF)],
                            out_hbm.at[h0 + j, pl.ds(col, BLOCK)],
                            ssem).start()

    def drain_store(j):
      pltpu.make_async_copy(rows_v.at[lax.rem(j, NBUF)],
                            out_hbm.at[h0 + j, pl.ds(col, BLOCK)],
                            ssem).wait()

    for j in range(GAHEAD):
      fill(j)

    for j in range(NBUF - GAHEAD):
      fill(j + GAHEAD)
      drain_gather(j)
      start_store(j)

    def step(j, carry):
      drain_store(j - (NBUF - GAHEAD))
      fill(j + GAHEAD)
      drain_gather(j)
      start_store(j)
      return carry

    lax.fori_loop(NBUF - GAHEAD, BAND - GAHEAD, step, 0)

    for j in range(BAND - GAHEAD, BAND):
      drain_store(j - (NBUF - GAHEAD))
      drain_gather(j)
      start_store(j)
    for j in range(BAND - (NBUF - GAHEAD), BAND):
      drain_store(j)

  return gather_kernel


_gather = _make_gather()


def kernel(x, table):
  out = _gather(table, x.T.astype(jnp.int32))
  return out.transpose(1, 0, 2)
